# baseline (speedup 1.0000x reference)
import jax
import jax.numpy as jnp
from jax.experimental import pallas as pl
from jax.experimental.pallas import tpu as pltpu

_B = 4096
_EMB = 128
_ND = 5000
_NA = 8000
_HD = 2560
_HA = 4096
_B_TILE = 256
_NB = _B // _B_TILE


def _fused(rate_ref, d1_ref, d2_ref, a1_ref, a2_ref, er_ref, wd_ref, wa_ref,
           od_ref, oa_ref, out_ref):
    f32 = jnp.float32
    d1 = d1_ref[...].astype(f32)
    d2 = d2_ref[...].astype(f32)
    a1 = a1_ref[...].astype(f32)
    a2 = a2_ref[...].astype(f32)

    dn_t = (((1,), (1,)), ((), ()))
    dn_n = (((1,), (0,)), ((), ()))
    pd = (jax.lax.dot_general(d1, wd_ref[:, 0:_HD], dn_t, preferred_element_type=f32)
          + jax.lax.dot_general(d2, wd_ref[:, _HD:2 * _HD], dn_t, preferred_element_type=f32))
    pa = (jax.lax.dot_general(a1, wa_ref[:, 0:_HA], dn_t, preferred_element_type=f32)
          + jax.lax.dot_general(a2, wa_ref[:, _HA:2 * _HA], dn_t, preferred_element_type=f32))
    sd = (jax.lax.dot_general(d1, od_ref[0:_HD], dn_n, preferred_element_type=f32)
          + jax.lax.dot_general(d2, od_ref[_HD:2 * _HD], dn_n, preferred_element_type=f32))
    sa = (jax.lax.dot_general(a1, oa_ref[0:_HA], dn_n, preferred_element_type=f32)
          + jax.lax.dot_general(a2, oa_ref[_HA:2 * _HA], dn_n, preferred_element_type=f32))

    classes = jax.lax.broadcasted_iota(jnp.int32, (1, 8), 1)
    onehot = (rate_ref[...] == classes).astype(f32)
    out_ref[:, 0:_EMB] = jax.lax.dot_general(
        onehot, er_ref[...], dn_n, preferred_element_type=f32)
    out_ref[:, _EMB:2 * _EMB] = pd / sd[:, 0:1]
    out_ref[:, 2 * _EMB:3 * _EMB] = pa / sa[:, 0:1]


def kernel(rate_idx, director_idx, actors_idx, emb_rate, W_director, W_actor):
    rate2d = rate_idx.astype(jnp.int32).reshape(_B, 1)
    er_pad = jnp.pad(emb_rate, ((0, 2), (0, 0)))
    wd = jnp.pad(W_director, ((0, 0), (0, 2 * _HD - _ND)))
    wa = jnp.pad(W_actor, ((0, 0), (0, 2 * _HA - _NA)))
    ones_d = jnp.pad(jnp.ones((_ND, 8), jnp.float32), ((0, 2 * _HD - _ND), (0, 0)))
    ones_a = jnp.pad(jnp.ones((_NA, 8), jnp.float32), ((0, 2 * _HA - _NA), (0, 0)))

    return pl.pallas_call(
        _fused,
        grid=(_NB,),
        in_specs=[
            pl.BlockSpec((_B_TILE, 1), lambda i: (i, 0)),
            pl.BlockSpec((_B_TILE, _HD), lambda i: (i, 0)),
            pl.BlockSpec((_B_TILE, _HD), lambda i: (i, 1)),
            pl.BlockSpec((_B_TILE, _HA), lambda i: (i, 0)),
            pl.BlockSpec((_B_TILE, _HA), lambda i: (i, 1)),
            pl.BlockSpec((8, _EMB), lambda i: (0, 0)),
            pl.BlockSpec((_EMB, 2 * _HD), lambda i: (0, 0)),
            pl.BlockSpec((_EMB, 2 * _HA), lambda i: (0, 0)),
            pl.BlockSpec((2 * _HD, 8), lambda i: (0, 0)),
            pl.BlockSpec((2 * _HA, 8), lambda i: (0, 0)),
        ],
        out_specs=pl.BlockSpec((_B_TILE, 3 * _EMB), lambda i: (i, 0)),
        out_shape=jax.ShapeDtypeStruct((_B, 3 * _EMB), jnp.float32),
        compiler_params=pltpu.CompilerParams(
            dimension_semantics=("parallel",)),
    )(rate2d, director_idx, director_idx, actors_idx, actors_idx,
      er_pad, wd, wa, ones_d, ones_a)


# speedup vs baseline: 1.2115x; 1.2115x over previous
import jax
import jax.numpy as jnp
from jax.experimental import pallas as pl
from jax.experimental.pallas import tpu as pltpu

_B = 4096
_EMB = 128
_ND = 5000
_NA = 8000
_HD = 2560
_HA = 4096
_B_TILE = 256
_NB = _B // _B_TILE


def _fused(rate_ref, d1_ref, d2_ref, a1_ref, a2_ref, er_ref, wd_ref, wa_ref,
           od_ref, oa_ref, out_ref):
    f32 = jnp.float32
    d1 = d1_ref[...].astype(f32)
    d2 = d2_ref[...].astype(f32)
    a1 = a1_ref[...].astype(f32)
    a2 = a2_ref[...].astype(f32)

    dn_t = (((1,), (1,)), ((), ()))
    dn_n = (((1,), (0,)), ((), ()))
    pd = (jax.lax.dot_general(d1, wd_ref[:, 0:_HD], dn_t, preferred_element_type=f32)
          + jax.lax.dot_general(d2, wd_ref[:, _HD:2 * _HD], dn_t, preferred_element_type=f32))
    pa = (jax.lax.dot_general(a1, wa_ref[:, 0:_HA], dn_t, preferred_element_type=f32)
          + jax.lax.dot_general(a2, wa_ref[:, _HA:2 * _HA], dn_t, preferred_element_type=f32))
    sd = (jax.lax.dot_general(d1, od_ref[0:_HD], dn_n, preferred_element_type=f32)
          + jax.lax.dot_general(d2, od_ref[_HD:2 * _HD], dn_n, preferred_element_type=f32))
    sa = (jax.lax.dot_general(a1, oa_ref[0:_HA], dn_n, preferred_element_type=f32)
          + jax.lax.dot_general(a2, oa_ref[_HA:2 * _HA], dn_n, preferred_element_type=f32))

    classes = jax.lax.broadcasted_iota(jnp.int32, (1, 8), 1)
    onehot = (rate_ref[...] == classes).astype(f32)
    out_ref[:, 0:_EMB] = jax.lax.dot_general(
        onehot, er_ref[...], dn_n, preferred_element_type=f32)
    out_ref[:, _EMB:2 * _EMB] = pd / sd[:, 0:1]
    out_ref[:, 2 * _EMB:3 * _EMB] = pa / sa[:, 0:1]


def kernel(rate_idx, director_idx, actors_idx, emb_rate, W_director, W_actor):
    rate2d = rate_idx.astype(jnp.int32).reshape(_B, 1)
    d8 = director_idx.astype(jnp.int8)
    a8 = actors_idx.astype(jnp.int8)
    er_pad = jnp.pad(emb_rate, ((0, 2), (0, 0)))
    wd = jnp.pad(W_director, ((0, 0), (0, 2 * _HD - _ND)))
    wa = jnp.pad(W_actor, ((0, 0), (0, 2 * _HA - _NA)))
    ones_d = jnp.pad(jnp.ones((_ND, 8), jnp.float32), ((0, 2 * _HD - _ND), (0, 0)))
    ones_a = jnp.pad(jnp.ones((_NA, 8), jnp.float32), ((0, 2 * _HA - _NA), (0, 0)))

    return pl.pallas_call(
        _fused,
        grid=(_NB,),
        in_specs=[
            pl.BlockSpec((_B_TILE, 1), lambda i: (i, 0)),
            pl.BlockSpec((_B_TILE, _HD), lambda i: (i, 0)),
            pl.BlockSpec((_B_TILE, _HD), lambda i: (i, 1)),
            pl.BlockSpec((_B_TILE, _HA), lambda i: (i, 0)),
            pl.BlockSpec((_B_TILE, _HA), lambda i: (i, 1)),
            pl.BlockSpec((8, _EMB), lambda i: (0, 0)),
            pl.BlockSpec((_EMB, 2 * _HD), lambda i: (0, 0)),
            pl.BlockSpec((_EMB, 2 * _HA), lambda i: (0, 0)),
            pl.BlockSpec((2 * _HD, 8), lambda i: (0, 0)),
            pl.BlockSpec((2 * _HA, 8), lambda i: (0, 0)),
        ],
        out_specs=pl.BlockSpec((_B_TILE, 3 * _EMB), lambda i: (i, 0)),
        out_shape=jax.ShapeDtypeStruct((_B, 3 * _EMB), jnp.float32),
        compiler_params=pltpu.CompilerParams(
            dimension_semantics=("parallel",)),
    )(rate2d, d8, d8, a8, a8,
      er_pad, wd, wa, ones_d, ones_a)


# speedup vs baseline: 1.3050x; 1.0772x over previous
import jax
import jax.numpy as jnp
from jax.experimental import pallas as pl
from jax.experimental.pallas import tpu as pltpu

_B = 4096
_EMB = 128
_ND = 5000
_NA = 8000
_HD = 2560
_HA = 4096
_B_TILE = 256
_NB = _B // _B_TILE


def _fused(rate_ref, d1_ref, d2_ref, a1_ref, a2_ref, er_ref, wd_ref, wa_ref,
           od_ref, oa_ref, out_ref):
    f32 = jnp.float32
    d1 = d1_ref[...].astype(f32)
    d2 = d2_ref[...].astype(f32)
    a1 = a1_ref[...].astype(f32)
    a2 = a2_ref[...].astype(f32)

    dn_t = (((1,), (1,)), ((), ()))
    dn_n = (((1,), (0,)), ((), ()))
    pd = (jax.lax.dot_general(d1, wd_ref[:, 0:_HD], dn_t, preferred_element_type=f32)
          + jax.lax.dot_general(d2, wd_ref[:, _HD:2 * _HD], dn_t, preferred_element_type=f32))
    pa = (jax.lax.dot_general(a1, wa_ref[:, 0:_HA], dn_t, preferred_element_type=f32)
          + jax.lax.dot_general(a2, wa_ref[:, _HA:2 * _HA], dn_t, preferred_element_type=f32))
    sd = (jax.lax.dot_general(d1, od_ref[0:_HD], dn_n, preferred_element_type=f32)
          + jax.lax.dot_general(d2, od_ref[_HD:2 * _HD], dn_n, preferred_element_type=f32))
    sa = (jax.lax.dot_general(a1, oa_ref[0:_HA], dn_n, preferred_element_type=f32)
          + jax.lax.dot_general(a2, oa_ref[_HA:2 * _HA], dn_n, preferred_element_type=f32))

    classes = jax.lax.broadcasted_iota(jnp.int32, (1, 8), 1)
    onehot = (rate_ref[...] == classes).astype(f32)
    out_ref[:, 0:_EMB] = jax.lax.dot_general(
        onehot, er_ref[...], dn_n, preferred_element_type=f32)
    out_ref[:, _EMB:2 * _EMB] = pd / sd[:, 0:1]
    out_ref[:, 2 * _EMB:3 * _EMB] = pa / sa[:, 0:1]


def kernel(rate_idx, director_idx, actors_idx, emb_rate, W_director, W_actor):
    rate2d = rate_idx.astype(jnp.int32).reshape(_B, 1)
    d8 = director_idx.astype(jnp.int4)
    a8 = actors_idx.astype(jnp.int4)
    er_pad = jnp.pad(emb_rate, ((0, 2), (0, 0)))
    wd = jnp.pad(W_director, ((0, 0), (0, 2 * _HD - _ND)))
    wa = jnp.pad(W_actor, ((0, 0), (0, 2 * _HA - _NA)))
    ones_d = jnp.pad(jnp.ones((_ND, 8), jnp.float32), ((0, 2 * _HD - _ND), (0, 0)))
    ones_a = jnp.pad(jnp.ones((_NA, 8), jnp.float32), ((0, 2 * _HA - _NA), (0, 0)))

    return pl.pallas_call(
        _fused,
        grid=(_NB,),
        in_specs=[
            pl.BlockSpec((_B_TILE, 1), lambda i: (i, 0)),
            pl.BlockSpec((_B_TILE, _HD), lambda i: (i, 0)),
            pl.BlockSpec((_B_TILE, _HD), lambda i: (i, 1)),
            pl.BlockSpec((_B_TILE, _HA), lambda i: (i, 0)),
            pl.BlockSpec((_B_TILE, _HA), lambda i: (i, 1)),
            pl.BlockSpec((8, _EMB), lambda i: (0, 0)),
            pl.BlockSpec((_EMB, 2 * _HD), lambda i: (0, 0)),
            pl.BlockSpec((_EMB, 2 * _HA), lambda i: (0, 0)),
            pl.BlockSpec((2 * _HD, 8), lambda i: (0, 0)),
            pl.BlockSpec((2 * _HA, 8), lambda i: (0, 0)),
        ],
        out_specs=pl.BlockSpec((_B_TILE, 3 * _EMB), lambda i: (i, 0)),
        out_shape=jax.ShapeDtypeStruct((_B, 3 * _EMB), jnp.float32),
        compiler_params=pltpu.CompilerParams(
            dimension_semantics=("parallel",)),
    )(rate2d, d8, d8, a8, a8,
      er_pad, wd, wa, ones_d, ones_a)
